# baseline (speedup 1.0000x reference)
import functools

import jax
import jax.numpy as jnp
from jax.experimental import pallas as pl
from jax.experimental.pallas import tpu as pltpu


def _agg_kernel(src_ref, nbr_a_ref, nbr_b_ref, w_ref, out_ref, *, k):
    s = src_ref.shape[0]
    d = src_ref.shape[1]
    h = s // 2
    nsum_a = jnp.reshape(nbr_a_ref[...], (h, k, d)).sum(axis=1)
    nsum_b = jnp.reshape(nbr_b_ref[...], (h, k, d)).sum(axis=1)
    nsum = jnp.concatenate([nsum_a, nsum_b], axis=0)
    mean = (nsum + src_ref[...]) * (1.0 / (k + 1))
    out_ref[...] = jax.nn.relu(
        jnp.dot(mean, w_ref[...], preferred_element_type=jnp.float32)
    )


def kernel(src_vectors, neighbor_vectors, W):
    n_src, d = src_vectors.shape
    n_nbr = neighbor_vectors.shape[0]
    out_dim = W.shape[1]
    k = n_nbr // n_src

    S = 400
    grid = (n_src // S,)
    hk = S * k // 2

    return pl.pallas_call(
        functools.partial(_agg_kernel, k=k),
        grid=grid,
        in_specs=[
            pl.BlockSpec((S, d), lambda i: (i, 0)),
            pl.BlockSpec((hk, d), lambda i: (2 * i, 0)),
            pl.BlockSpec((hk, d), lambda i: (2 * i + 1, 0)),
            pl.BlockSpec((d, out_dim), lambda i: (0, 0)),
        ],
        out_specs=pl.BlockSpec((S, out_dim), lambda i: (i, 0)),
        out_shape=jax.ShapeDtypeStruct((n_src, out_dim), jnp.float32),
        compiler_params=pltpu.CompilerParams(
            dimension_semantics=("parallel",),
        ),
    )(src_vectors, neighbor_vectors, neighbor_vectors, W)


# speedup vs baseline: 1.0128x; 1.0128x over previous
import functools

import jax
import jax.numpy as jnp
from jax.experimental import pallas as pl
from jax.experimental.pallas import tpu as pltpu


def _agg_kernel(src_ref, nbr_ref, w_ref, out_ref, *, k):
    s = src_ref.shape[0]
    d = src_ref.shape[1]
    nbr = nbr_ref[...]
    nsum = jnp.reshape(nbr, (s, k, d)).sum(axis=1)
    mean = (nsum + src_ref[...]) * (1.0 / (k + 1))
    out_ref[...] = jax.nn.relu(
        jnp.dot(mean, w_ref[...], preferred_element_type=jnp.float32)
    )


def kernel(src_vectors, neighbor_vectors, W):
    n_src, d = src_vectors.shape
    n_nbr = neighbor_vectors.shape[0]
    out_dim = W.shape[1]
    k = n_nbr // n_src

    S = 400
    grid = (n_src // S,)

    return pl.pallas_call(
        functools.partial(_agg_kernel, k=k),
        grid=grid,
        in_specs=[
            pl.BlockSpec((S, d), lambda i: (i, 0)),
            pl.BlockSpec((S * k, d), lambda i: (i, 0)),
            pl.BlockSpec((d, out_dim), lambda i: (0, 0)),
        ],
        out_specs=pl.BlockSpec((S, out_dim), lambda i: (i, 0)),
        out_shape=jax.ShapeDtypeStruct((n_src, out_dim), jnp.float32),
        compiler_params=pltpu.CompilerParams(
            dimension_semantics=("parallel",),
        ),
    )(src_vectors, neighbor_vectors, W)
